# Initial kernel scaffold; baseline (speedup 1.0000x reference)
import jax
import jax.numpy as jnp
from jax.experimental import pallas as pl


def kernel(class_logits, box_regression, box_ctrness, anchors):
    raise NotImplementedError("write your pallas kernel here")


# speedup vs baseline: 4.6183x; 4.6183x over previous
import jax
import jax.numpy as jnp
from jax import lax
from jax.experimental import pallas as pl
from jax.experimental.pallas import tpu as pltpu

_N = 87296
_C = 91
_R = 341
_L = 256
_BN = 2816
_G = 31
_IOU = 0.6
_K = 100


def _cls_kernel(logits_ref, ctr_ref, mx_ref, lbl_ref):
    lg = logits_ref[...]
    ct = ctr_ref[...]
    s = jnp.sqrt(jax.nn.sigmoid(lg) * jax.nn.sigmoid(ct))
    mx = jnp.max(s, axis=1, keepdims=True)
    lane = lax.broadcasted_iota(jnp.int32, s.shape, 1)
    lbl = jnp.min(jnp.where(s == mx, lane, _C), axis=1, keepdims=True)
    mx_ref[...] = mx
    lbl_ref[...] = lbl


def _nms_kernel(anc_ref, reg_ref, sc_ref, lb_ref, out_ref,
                x1s, y1s, x2s, y2s, ars, scs):
    ax1 = anc_ref[0]
    ay1 = anc_ref[1]
    ax2 = anc_ref[2]
    ay2 = anc_ref[3]
    cx = 0.5 * (ax1 + ax2)
    cy = 0.5 * (ay1 + ay2)
    w = ax2 - ax1
    h = ay2 - ay1
    x1 = cx - reg_ref[0] * w
    y1 = cy - reg_ref[1] * h
    x2 = cx + reg_ref[2] * w
    y2 = cy + reg_ref[3] * h
    x1s[...] = x1
    y1s[...] = y1
    x2s[...] = x2
    y2s[...] = y2
    ars[...] = jnp.maximum(x2 - x1, 0.0) * jnp.maximum(y2 - y1, 0.0)
    scs[...] = sc_ref[...]

    neginf = jnp.float32(-jnp.inf)

    def body(i, carry):
        s = scs[...]
        m = jnp.max(jnp.max(s, axis=0, keepdims=True), axis=1, keepdims=True)
        flat = (lax.broadcasted_iota(jnp.int32, s.shape, 0) * _L
                + lax.broadcasted_iota(jnp.int32, s.shape, 1))
        idxv = jnp.where(s == m, flat, _N)
        idxv = jnp.min(jnp.min(idxv, axis=0, keepdims=True), axis=1,
                       keepdims=True)
        idx = idxv[0, 0]
        r = lax.shift_right_logical(idx, 8)
        c = lax.bitwise_and(idx, 255)
        lane = lax.broadcasted_iota(jnp.int32, (1, _L), 1)
        sel = lane == c

        def pickf(ref):
            row = ref[pl.ds(r, 1), :]
            return jnp.max(jnp.where(sel, row, neginf), axis=1, keepdims=True)

        bx1 = pickf(x1s)
        by1 = pickf(y1s)
        bx2 = pickf(x2s)
        by2 = pickf(y2s)
        bar = pickf(ars)
        bsc = pickf(sc_ref)
        lrow = lb_ref[pl.ds(r, 1), :]
        blb = jnp.max(jnp.where(sel, lrow, -1), axis=1, keepdims=True)

        iw = jnp.maximum(jnp.minimum(x2s[...], bx2)
                         - jnp.maximum(x1s[...], bx1), 0.0)
        ih = jnp.maximum(jnp.minimum(y2s[...], by2)
                         - jnp.maximum(y1s[...], by1), 0.0)
        inter = iw * ih
        iou = inter / (bar + ars[...] - inter)
        scs[...] = jnp.where(iou > _IOU, neginf, s)

        li = lax.broadcasted_iota(jnp.int32, (1, 128), 1)
        v = jnp.where(
            li == 0, bx1,
            jnp.where(li == 1, by1,
                      jnp.where(li == 2, bx2,
                                jnp.where(li == 3, by2,
                                          jnp.where(li == 4, bsc,
                                                    jnp.where(li == 5,
                                                              blb.astype(jnp.float32),
                                                              0.0))))))
        out_ref[pl.ds(i, 1)] = v.reshape(1, 1, 128)
        return carry

    lax.fori_loop(0, _K, body, 0)


def _cls_call(class_logits, box_ctrness):
    return pl.pallas_call(
        _cls_kernel,
        grid=(_G,),
        in_specs=[
            pl.BlockSpec((_BN, _C), lambda i: (i, 0)),
            pl.BlockSpec((_BN, 1), lambda i: (i, 0)),
        ],
        out_specs=[
            pl.BlockSpec((_BN, 1), lambda i: (i, 0)),
            pl.BlockSpec((_BN, 1), lambda i: (i, 0)),
        ],
        out_shape=[
            jax.ShapeDtypeStruct((_N, 1), jnp.float32),
            jax.ShapeDtypeStruct((_N, 1), jnp.int32),
        ],
        compiler_params=pltpu.CompilerParams(
            dimension_semantics=("parallel",),
        ),
        name="fcos_cls_scores",
    )(class_logits, box_ctrness)


def _nms_call(anc_p, reg_p, sc_p, lb_p):
    return pl.pallas_call(
        _nms_kernel,
        out_shape=jax.ShapeDtypeStruct((_K, 1, 128), jnp.float32),
        scratch_shapes=[pltpu.VMEM((_R, _L), jnp.float32)] * 6,
        name="fcos_nms",
    )(anc_p, reg_p, sc_p, lb_p)


def kernel(class_logits, box_regression, box_ctrness, anchors):
    mx, lbl = _cls_call(class_logits, box_ctrness)
    anc_p = anchors.T.reshape(4, _R, _L)
    reg_p = box_regression.T.reshape(4, _R, _L)
    sc_p = mx.reshape(_R, _L)
    lb_p = lbl.reshape(_R, _L)
    out = _nms_call(anc_p, reg_p, sc_p, lb_p).reshape(_K, 128)
    pred_boxes = out[:, :4]
    scores = out[:, 4]
    labels = out[:, 5].astype(jnp.int32)
    return pred_boxes, labels, scores
